# Initial kernel scaffold; baseline (speedup 1.0000x reference)
import jax
import jax.numpy as jnp
from jax.experimental import pallas as pl


def kernel(input_ids, embed_table):
    raise NotImplementedError("write your pallas kernel here")


# speedup vs baseline: 1.5068x; 1.5068x over previous
import functools

import jax
import jax.numpy as jnp
from jax import lax
from jax.experimental import pallas as pl
from jax.experimental.pallas import tpu as pltpu
from jax.experimental.pallas import tpu_sc as plsc

import numpy as np

VOCAB = 100000
D_MODEL = 2048
HEAD_DIM = 128
THETA = 10000.0
B = 2
S = 4096
N_TOK = B * S

NC = 2
NS = 16
NW = NC * NS
TOK_PER_W = N_TOK // NW
CHUNK = 32
N_CHUNKS = TOK_PER_W // CHUNK


def _gather_body(ids_hbm, table_hbm, out_hbm, idx_v, rows_v, sem):
    wid = lax.axis_index("s") * NC + lax.axis_index("c")
    row0 = wid * N_CHUNKS
    pltpu.sync_copy(ids_hbm.at[pl.ds(row0, N_CHUNKS)], idx_v)
    for j in range(N_CHUNKS):
        pltpu.async_copy(table_hbm.at[idx_v.at[j]], rows_v, sem).wait()
        pltpu.sync_copy(
            rows_v, out_hbm.at[pl.ds(wid * TOK_PER_W + j * CHUNK, CHUNK)]
        )


@jax.jit
def _sc_gather(ids2d, table):
    mesh = plsc.VectorSubcoreMesh(core_axis_name="c", subcore_axis_name="s")
    f = pl.kernel(
        _gather_body,
        out_type=jax.ShapeDtypeStruct((N_TOK, D_MODEL), jnp.float32),
        mesh=mesh,
        scratch_types=[
            pltpu.VMEM((N_CHUNKS, CHUNK), jnp.int32),
            pltpu.VMEM((CHUNK, D_MODEL), jnp.float32),
            pltpu.SemaphoreType.DMA,
        ],
    )
    return f(ids2d, table)


def _rope_body(cos_ref, sin_ref):
    pos = lax.broadcasted_iota(jnp.int32, (S, HEAD_DIM), 0).astype(jnp.float32)
    col = lax.broadcasted_iota(jnp.int32, (S, HEAD_DIM), 1)
    half = jnp.where(col < HEAD_DIM // 2, col, col - HEAD_DIM // 2)
    log_theta = float(np.log(THETA))
    inv_freq = jnp.exp(half.astype(jnp.float32) * (-2.0 / HEAD_DIM * log_theta))
    ang = pos * inv_freq
    cos_ref[...] = jnp.cos(ang)
    sin_ref[...] = jnp.sin(ang)


@jax.jit
def _rope():
    return pl.pallas_call(
        _rope_body,
        out_shape=(
            jax.ShapeDtypeStruct((S, HEAD_DIM), jnp.float32),
            jax.ShapeDtypeStruct((S, HEAD_DIM), jnp.float32),
        ),
    )()


def kernel(input_ids, embed_table):
    ids2d = input_ids.reshape(NW * N_CHUNKS, CHUNK)
    hid = _sc_gather(ids2d, embed_table)
    cos, sin = _rope()
    return (
        hid.reshape(B, S, D_MODEL),
        cos[None],
        sin[None],
    )


# speedup vs baseline: 1.5764x; 1.0462x over previous
import functools

import jax
import jax.numpy as jnp
from jax import lax
from jax.experimental import pallas as pl
from jax.experimental.pallas import tpu as pltpu
from jax.experimental.pallas import tpu_sc as plsc

import numpy as np

VOCAB = 100000
D_MODEL = 2048
HEAD_DIM = 128
THETA = 10000.0
B = 2
S = 4096
N_TOK = B * S

NC = 2
NS = 16
NW = NC * NS
TOK_PER_W = N_TOK // NW
CHUNK = 16
N_CHUNKS = TOK_PER_W // CHUNK


def _gather_body(ids_hbm, table_hbm, out_hbm, idx_v, bufs, sem):
    wid = lax.axis_index("s") * NC + lax.axis_index("c")
    pltpu.sync_copy(ids_hbm.at[pl.ds(wid * N_CHUNKS, N_CHUNKS)], idx_v)

    def start_gather(j, b):
        pltpu.async_copy(table_hbm.at[idx_v.at[j]], bufs.at[b], sem)

    def wait_gather(j, b):
        pltpu.make_async_copy(table_hbm.at[idx_v.at[j]], bufs.at[b], sem).wait()

    def store(j, b):
        pltpu.sync_copy(
            bufs.at[b], out_hbm.at[pl.ds(wid * TOK_PER_W + j * CHUNK, CHUNK)]
        )

    start_gather(0, 0)

    @pl.loop(0, N_CHUNKS, step=2)
    def _(j):
        wait_gather(j, 0)
        start_gather(j + 1, 1)
        store(j, 0)
        wait_gather(j + 1, 1)

        @pl.when(j + 2 < N_CHUNKS)
        def _():
            start_gather(j + 2, 0)

        store(j + 1, 1)


@jax.jit
def _sc_gather(ids2d, table):
    mesh = plsc.VectorSubcoreMesh(core_axis_name="c", subcore_axis_name="s")
    f = pl.kernel(
        _gather_body,
        out_type=jax.ShapeDtypeStruct((N_TOK, D_MODEL), jnp.float32),
        mesh=mesh,
        scratch_types=[
            pltpu.VMEM((N_CHUNKS, CHUNK), jnp.int32),
            pltpu.VMEM((2, CHUNK, D_MODEL), jnp.float32),
            pltpu.SemaphoreType.DMA,
        ],
    )
    return f(ids2d, table)


def _rope_body(cos_ref, sin_ref):
    pos = lax.broadcasted_iota(jnp.int32, (S, HEAD_DIM), 0).astype(jnp.float32)
    col = lax.broadcasted_iota(jnp.int32, (S, HEAD_DIM), 1)
    half = jnp.where(col < HEAD_DIM // 2, col, col - HEAD_DIM // 2)
    log_theta = float(np.log(THETA))
    inv_freq = jnp.exp(half.astype(jnp.float32) * (-2.0 / HEAD_DIM * log_theta))
    ang = pos * inv_freq
    cos_ref[...] = jnp.cos(ang)
    sin_ref[...] = jnp.sin(ang)


@jax.jit
def _rope():
    return pl.pallas_call(
        _rope_body,
        out_shape=(
            jax.ShapeDtypeStruct((S, HEAD_DIM), jnp.float32),
            jax.ShapeDtypeStruct((S, HEAD_DIM), jnp.float32),
        ),
    )()


def kernel(input_ids, embed_table):
    ids2d = input_ids.reshape(NW * N_CHUNKS, CHUNK)
    hid = _sc_gather(ids2d, embed_table)
    cos, sin = _rope()
    return (
        hid.reshape(B, S, D_MODEL),
        cos[None],
        sin[None],
    )


# speedup vs baseline: 1.6130x; 1.0232x over previous
import functools

import jax
import jax.numpy as jnp
from jax import lax
from jax.experimental import pallas as pl
from jax.experimental.pallas import tpu as pltpu
from jax.experimental.pallas import tpu_sc as plsc

import numpy as np

VOCAB = 100000
D_MODEL = 2048
HEAD_DIM = 128
THETA = 10000.0
B = 2
S = 4096
N_TOK = B * S

NC = 2
NS = 16
NW = NC * NS
TOK_PER_W = N_TOK // NW
CHUNK = 8
N_CHUNKS = TOK_PER_W // CHUNK
NBUF = 4


def _gather_body(ids_hbm, table_hbm, out_hbm, idx_v, bufs, *sems):
    sem_in, sem_out = sems[:NBUF], sems[NBUF:]
    wid = lax.axis_index("s") * NC + lax.axis_index("c")
    pltpu.sync_copy(ids_hbm.at[pl.ds(wid * N_CHUNKS, N_CHUNKS)], idx_v)

    def gather_desc(j, b):
        return pltpu.make_async_copy(
            table_hbm.at[idx_v.at[j]], bufs.at[b], sem_in[b]
        )

    def store_desc(j, b):
        return pltpu.make_async_copy(
            bufs.at[b], out_hbm.at[pl.ds(wid * TOK_PER_W + j * CHUNK, CHUNK)],
            sem_out[b],
        )

    for b in range(NBUF - 1):
        gather_desc(b, b).start()

    @pl.loop(0, N_CHUNKS, step=NBUF)
    def _(j):
        for b in range(NBUF):
            jj = j + b
            gather_desc(jj, b).wait()
            store_desc(jj, b).start()
            bf = (b + NBUF - 1) % NBUF

            @pl.when(jj + NBUF - 1 < N_CHUNKS)
            def _():
                @pl.when(jj > 0)
                def _():
                    store_desc(jj - 1, bf).wait()

                gather_desc(jj + NBUF - 1, bf).start()

    for jj in range(N_CHUNKS - NBUF, N_CHUNKS):
        store_desc(jj, jj % NBUF).wait()


@jax.jit
def _sc_gather(ids2d, table):
    mesh = plsc.VectorSubcoreMesh(core_axis_name="c", subcore_axis_name="s")
    f = pl.kernel(
        _gather_body,
        out_type=jax.ShapeDtypeStruct((N_TOK, D_MODEL), jnp.float32),
        mesh=mesh,
        scratch_types=[
            pltpu.VMEM((N_CHUNKS, CHUNK), jnp.int32),
            pltpu.VMEM((NBUF, CHUNK, D_MODEL), jnp.float32),
        ] + [pltpu.SemaphoreType.DMA] * (2 * NBUF),
    )
    return f(ids2d, table)


def _rope_body(cos_ref, sin_ref):
    pos = lax.broadcasted_iota(jnp.int32, (S, HEAD_DIM), 0).astype(jnp.float32)
    col = lax.broadcasted_iota(jnp.int32, (S, HEAD_DIM), 1)
    half = jnp.where(col < HEAD_DIM // 2, col, col - HEAD_DIM // 2)
    log_theta = float(np.log(THETA))
    inv_freq = jnp.exp(half.astype(jnp.float32) * (-2.0 / HEAD_DIM * log_theta))
    ang = pos * inv_freq
    cos_ref[...] = jnp.cos(ang)
    sin_ref[...] = jnp.sin(ang)


@jax.jit
def _rope():
    return pl.pallas_call(
        _rope_body,
        out_shape=(
            jax.ShapeDtypeStruct((S, HEAD_DIM), jnp.float32),
            jax.ShapeDtypeStruct((S, HEAD_DIM), jnp.float32),
        ),
    )()


def kernel(input_ids, embed_table):
    ids2d = input_ids.reshape(NW * N_CHUNKS, CHUNK)
    hid = _sc_gather(ids2d, embed_table)
    cos, sin = _rope()
    return (
        hid.reshape(B, S, D_MODEL),
        cos[None],
        sin[None],
    )
